# baseline (speedup 1.0000x reference)
import functools

import jax
import jax.numpy as jnp
from jax import lax
from jax.experimental import pallas as pl
from jax.experimental.pallas import tpu as pltpu
from jax.experimental.pallas import tpu_sc as plsc

N = 10000
E = 320000
IN_CH = 128
HID = 32

NC = 2
NS = 16
NW = NC * NS
EPW = E // NW
CH = 1000
NCHUNK = EPW // CH
RA = 624
RLAST = N - (NS - 1) * RA
CNTW = 16

_MESH = plsc.VectorSubcoreMesh(
    core_axis_name="c", subcore_axis_name="s", num_cores=NC, num_subcores=NS
)
_SC_PARAMS = pltpu.CompilerParams(use_tc_tiling_on_sc=False)


def _f32(*shape):
    return jax.ShapeDtypeStruct(shape, jnp.float32)


def _tile_layout(sid):
    row0 = pl.multiple_of(sid * RA, 8)
    is_last = sid == NS - 1
    return row0, is_last


def _per_tile(is_last, fn):
    @pl.when(is_last)
    def _():
        fn(RLAST)

    @pl.when(jnp.logical_not(is_last))
    def _():
        fn(RA)


def _edge_pipeline(ei_hbm, wid, bufs, gather_tab, scatter_fn):
    def load_idx(g, sv, dv):
        base = pl.multiple_of(wid * EPW + g * CH, 8)
        pltpu.sync_copy(ei_hbm.at[0, pl.ds(base, CH)], sv)
        pltpu.sync_copy(ei_hbm.at[1, pl.ds(base, CH)], dv)

    src0, dst0 = bufs[0][0], bufs[0][1]
    load_idx(0, src0, dst0)
    pltpu.make_async_copy(gather_tab.at[src0], bufs[0][2], bufs[0][3]).start()

    def pair_body(i, _):
        for b in range(2):
            g = 2 * i + b
            sv, dv, rv, sm = bufs[b]
            sv2, dv2, rv2, sm2 = bufs[1 - b]

            def prefetch():
                load_idx(g + 1, sv2, dv2)
                pltpu.make_async_copy(gather_tab.at[sv2], rv2, sm2).start()

            if b == 0:
                prefetch()
            else:
                @pl.when(i < NCHUNK // 2 - 1)
                def _():
                    prefetch()

            pltpu.make_async_copy(gather_tab.at[sv], rv, sm).wait()
            scatter_fn(rv, dv)
        return 0

    lax.fori_loop(0, NCHUNK // 2, pair_body, 0)


def _copy_rows(src_at, dst_at, is_last):
    def go(nr):
        pltpu.sync_copy(src_at(nr), dst_at(nr))
    _per_tile(is_last, go)


def _sc_pass1(edge_index, y1):
    scratch = [
        pltpu.VMEM((CH,), jnp.int32),
        pltpu.VMEM((CH,), jnp.int32),
        pltpu.VMEM((CH,), jnp.int32),
        pltpu.VMEM((CH,), jnp.int32),
        pltpu.VMEM((CH, HID), jnp.float32),
        pltpu.VMEM((CH, HID), jnp.float32),
        pltpu.SemaphoreType.DMA,
        pltpu.SemaphoreType.DMA,
        pltpu.VMEM((CH, CNTW), jnp.float32),
        pltpu.VMEM_SHARED((N, HID), jnp.float32),
        pltpu.VMEM_SHARED((N, CNTW), jnp.float32),
    ]

    def body(ei_hbm, y_hbm, acc_out, cnt_out, src0, src1, dst0, dst1,
             rows0, rows1, sem0, sem1, ones_v, acc_sh, cnt_sh):
        cid = lax.axis_index("c")
        sid = lax.axis_index("s")
        wid = sid * NC + cid
        row0, is_last = _tile_layout(sid)

        def zrows(i, _):
            rows0[i, pl.ds(0, 16)] = jnp.zeros((16,), jnp.float32)
            rows0[i, pl.ds(16, 16)] = jnp.zeros((16,), jnp.float32)
            ones_v[i, pl.ds(0, 16)] = jnp.zeros((16,), jnp.float32)
            return 0

        lax.fori_loop(0, RLAST, zrows, 0)
        _copy_rows(lambda nr: rows0.at[pl.ds(0, nr)],
                   lambda nr: acc_sh.at[pl.ds(row0, nr)], is_last)
        _copy_rows(lambda nr: ones_v.at[pl.ds(0, nr)],
                   lambda nr: cnt_sh.at[pl.ds(row0, nr)], is_last)

        def fill_ones(i, _):
            ones_v[i, pl.ds(0, 16)] = jnp.ones((16,), jnp.float32)
            return 0

        lax.fori_loop(0, CH, fill_ones, 0)
        plsc.subcore_barrier()

        bufs = ((src0, dst0, rows0, sem0), (src1, dst1, rows1, sem1))

        def scatter(rv, dv):
            pltpu.sync_copy(rv, acc_sh.at[dv], add=True)
            pltpu.sync_copy(ones_v, cnt_sh.at[dv], add=True)

        _edge_pipeline(ei_hbm, wid, bufs, y_hbm, scatter)
        plsc.subcore_barrier()

        _copy_rows(lambda nr: acc_sh.at[pl.ds(row0, nr)],
                   lambda nr: acc_out.at[cid, pl.ds(row0, nr)], is_last)
        _copy_rows(lambda nr: cnt_sh.at[pl.ds(row0, nr)],
                   lambda nr: cnt_out.at[cid, pl.ds(row0, nr)], is_last)

    run = pl.kernel(
        body,
        out_type=[_f32(NC, N, HID), _f32(NC, N, CNTW)],
        mesh=_MESH,
        scratch_types=scratch, compiler_params=_SC_PARAMS,
        name="sage_sc_pass1",
    )
    return run(edge_index, y1)


def _sc_pass2(edge_index, acc1, cntp, xrb):
    scratch = [
        pltpu.VMEM((CH,), jnp.int32),
        pltpu.VMEM((CH,), jnp.int32),
        pltpu.VMEM((CH,), jnp.int32),
        pltpu.VMEM((CH,), jnp.int32),
        pltpu.VMEM((CH, HID), jnp.float32),
        pltpu.VMEM((CH, HID), jnp.float32),
        pltpu.SemaphoreType.DMA,
        pltpu.SemaphoreType.DMA,
        pltpu.VMEM((RLAST, CNTW), jnp.float32),
        pltpu.VMEM((RLAST, CNTW), jnp.float32),
        pltpu.VMEM((RLAST, HID), jnp.float32),
        pltpu.VMEM_SHARED((N, HID), jnp.float32),
    ]

    def body(ei_hbm, acc1_hbm, cntp_hbm, xrb_hbm, acc_out, h_out,
             src0, src1, dst0, dst1, rows0, rows1, sem0, sem1,
             cb0, cb1, xb, acc_sh):
        cid = lax.axis_index("c")
        sid = lax.axis_index("s")
        wid = sid * NC + cid
        row0, is_last = _tile_layout(sid)

        def stage(nr):
            ds = [
                pltpu.make_async_copy(acc1_hbm.at[0, pl.ds(row0, nr)],
                                      rows0.at[pl.ds(0, nr)], sem0),
                pltpu.make_async_copy(acc1_hbm.at[1, pl.ds(row0, nr)],
                                      rows1.at[pl.ds(0, nr)], sem0),
                pltpu.make_async_copy(cntp_hbm.at[0, pl.ds(row0, nr)],
                                      cb0.at[pl.ds(0, nr)], sem0),
                pltpu.make_async_copy(cntp_hbm.at[1, pl.ds(row0, nr)],
                                      cb1.at[pl.ds(0, nr)], sem0),
                pltpu.make_async_copy(xrb_hbm.at[pl.ds(row0, nr)],
                                      xb.at[pl.ds(0, nr)], sem0),
            ]
            for d in ds:
                d.start()
            for d in ds:
                d.wait()

        _per_tile(is_last, stage)

        def hrow(r, _):
            cv = jnp.maximum(cb0[r, pl.ds(0, 16)] + cb1[r, pl.ds(0, 16)], 1.0)
            inv = 1.0 / cv
            for j in range(HID // 16):
                sl = pl.ds(j * 16, 16)
                v = (rows0[r, sl] + rows1[r, sl]) * inv + xb[r, sl]
                xb[r, sl] = jnp.maximum(v, 0.0)
            return 0

        def hcompute(nr):
            lax.fori_loop(0, nr, hrow, 0)
        _per_tile(is_last, hcompute)

        _copy_rows(lambda nr: xb.at[pl.ds(0, nr)],
                   lambda nr: h_out.at[cid, pl.ds(row0, nr)], is_last)

        def zrows(i, _):
            rows0[i, pl.ds(0, 16)] = jnp.zeros((16,), jnp.float32)
            rows0[i, pl.ds(16, 16)] = jnp.zeros((16,), jnp.float32)
            return 0

        lax.fori_loop(0, RLAST, zrows, 0)
        _copy_rows(lambda nr: rows0.at[pl.ds(0, nr)],
                   lambda nr: acc_sh.at[pl.ds(row0, nr)], is_last)
        plsc.subcore_barrier()

        bufs = ((src0, dst0, rows0, sem0), (src1, dst1, rows1, sem1))

        def scatter(rv, dv):
            pltpu.sync_copy(rv, acc_sh.at[dv], add=True)

        _edge_pipeline(ei_hbm, wid, bufs, h_out.at[cid], scatter)
        plsc.subcore_barrier()

        def divrow(r, _):
            cv = jnp.maximum(cb0[r, pl.ds(0, 16)] + cb1[r, pl.ds(0, 16)], 1.0)
            inv = 1.0 / cv
            for j in range(HID // 16):
                sl = pl.ds(j * 16, 16)
                rows0[r, sl] = rows0[r, sl] * inv
            return 0

        def divphase(nr):
            pltpu.sync_copy(acc_sh.at[pl.ds(row0, nr)], rows0.at[pl.ds(0, nr)])
            lax.fori_loop(0, nr, divrow, 0)
            pltpu.sync_copy(rows0.at[pl.ds(0, nr)],
                            acc_out.at[cid, pl.ds(row0, nr)])

        _per_tile(is_last, divphase)

    run = pl.kernel(
        body,
        out_type=[_f32(NC, N, HID), _f32(NC, N, HID)],
        mesh=_MESH, scratch_types=scratch, compiler_params=_SC_PARAMS,
        name="sage_sc_pass2",
    )
    return run(edge_index, acc1, cntp, xrb)


def _dotT(a, w):
    return lax.dot_general(a, w, (((1,), (1,)), ((), ())),
                           preferred_element_type=jnp.float32)


def _pre_body(xp_ref, w4l_ref, w4r_ref, b14_ref, y1p_ref, xrbp_ref):
    xv = xp_ref[...]
    y1p_ref[...] = jnp.dot(xv, w4l_ref[...],
                           preferred_element_type=jnp.float32)
    xrbp_ref[...] = jnp.dot(xv, w4r_ref[...],
                            preferred_element_type=jnp.float32) + b14_ref[...]


def _out_body(acc2p_ref, h0p_ref, w2l_ref, w2r_ref, b2_ref, out_ref):
    mean2 = acc2p_ref[0] + acc2p_ref[1]
    h = h0p_ref[0]
    wl = w2l_ref[...]
    wr = w2r_ref[...]
    b2 = b2_ref[...]
    for i in range(4):
        sl = slice(32 * i, 32 * i + 32)
        out_ref[:, sl] = (_dotT(mean2[:, sl], wl) + b2
                          + _dotT(h[:, sl], wr))


def kernel(x, edge_index, W1_l, b1_l, W1_r, W2_l, b2_l, W2_r):
    eye4 = jnp.eye(4, dtype=jnp.float32)
    W4l = jnp.kron(eye4, W1_l.T)
    W4r = jnp.kron(eye4, W1_r.T)
    b14 = jnp.tile(b1_l, 4).reshape(1, 128)

    y1p, xrbp = pl.pallas_call(
        _pre_body,
        out_shape=[_f32(N // 4, 128), _f32(N // 4, 128)],
    )(x.reshape(N // 4, 4 * IN_CH), W4l, W4r, b14)
    y1 = y1p.reshape(N, HID)
    xrb = xrbp.reshape(N, HID)

    acc1, cntp = _sc_pass1(edge_index, y1)
    acc2, h2 = _sc_pass2(edge_index, acc1, cntp, xrb)

    acc2p = acc2.reshape(NC, N // 4, 128)
    h2p = h2.reshape(NC, N // 4, 128)

    outp = pl.pallas_call(
        _out_body,
        out_shape=_f32(N // 4, 128),
        grid=(1,),
        in_specs=[
            pl.BlockSpec((NC, N // 4, 128), lambda i: (0, 0, 0)),
            pl.BlockSpec((1, N // 4, 128), lambda i: (0, 0, 0)),
            pl.BlockSpec((HID, HID), lambda i: (0, 0)),
            pl.BlockSpec((HID, HID), lambda i: (0, 0)),
            pl.BlockSpec((1, HID), lambda i: (0, 0)),
        ],
        out_specs=pl.BlockSpec((N // 4, 128), lambda i: (0, 0)),
    )(acc2p, h2p, W2_l, W2_r, b2_l.reshape(1, HID))

    return outp.reshape(N, HID)


# speedup vs baseline: 1.0082x; 1.0082x over previous
import functools

import jax
import jax.numpy as jnp
from jax import lax
from jax.experimental import pallas as pl
from jax.experimental.pallas import tpu as pltpu
from jax.experimental.pallas import tpu_sc as plsc

N = 10000
E = 320000
IN_CH = 128
HID = 32

NC = 2
NS = 16
NW = NC * NS
EPW = E // NW
CH = 1000
NCHUNK = EPW // CH
RA = 624
RLAST = N - (NS - 1) * RA
CNTW = 16

_MESH = plsc.VectorSubcoreMesh(
    core_axis_name="c", subcore_axis_name="s", num_cores=NC, num_subcores=NS
)
_SC_PARAMS = pltpu.CompilerParams(use_tc_tiling_on_sc=False)


def _f32(*shape):
    return jax.ShapeDtypeStruct(shape, jnp.float32)


def _tile_layout(sid):
    row0 = pl.multiple_of(sid * RA, 8)
    is_last = sid == NS - 1
    return row0, is_last


def _per_tile(is_last, fn):
    @pl.when(is_last)
    def _():
        fn(RLAST)

    @pl.when(jnp.logical_not(is_last))
    def _():
        fn(RA)


def _edge_pipeline(ei_hbm, wid, bufs, gather_tab, scatter_fn):
    def load_idx(g, sv, dv):
        base = pl.multiple_of(wid * EPW + g * CH, 8)
        pltpu.sync_copy(ei_hbm.at[0, pl.ds(base, CH)], sv)
        pltpu.sync_copy(ei_hbm.at[1, pl.ds(base, CH)], dv)

    src0, dst0 = bufs[0][0], bufs[0][1]
    load_idx(0, src0, dst0)
    pltpu.make_async_copy(gather_tab.at[src0], bufs[0][2], bufs[0][3]).start()

    def pair_body(i, _):
        for b in range(2):
            g = 2 * i + b
            sv, dv, rv, sm = bufs[b]
            sv2, dv2, rv2, sm2 = bufs[1 - b]

            def prefetch():
                load_idx(g + 1, sv2, dv2)
                pltpu.make_async_copy(gather_tab.at[sv2], rv2, sm2).start()

            if b == 0:
                prefetch()
            else:
                @pl.when(i < NCHUNK // 2 - 1)
                def _():
                    prefetch()

            pltpu.make_async_copy(gather_tab.at[sv], rv, sm).wait()
            scatter_fn(rv, dv)
        return 0

    lax.fori_loop(0, NCHUNK // 2, pair_body, 0)


def _copy_rows(src_at, dst_at, is_last):
    def go(nr):
        pltpu.sync_copy(src_at(nr), dst_at(nr))
    _per_tile(is_last, go)


def _sc_pass1(edge_index, y1):
    scratch = [
        pltpu.VMEM((CH,), jnp.int32),
        pltpu.VMEM((CH,), jnp.int32),
        pltpu.VMEM((CH,), jnp.int32),
        pltpu.VMEM((CH,), jnp.int32),
        pltpu.VMEM((CH, HID), jnp.float32),
        pltpu.VMEM((CH, HID), jnp.float32),
        pltpu.SemaphoreType.DMA,
        pltpu.SemaphoreType.DMA,
        pltpu.VMEM((CH, CNTW), jnp.float32),
        pltpu.VMEM_SHARED((N, HID), jnp.float32),
        pltpu.VMEM_SHARED((N, CNTW), jnp.float32),
    ]

    def body(ei_hbm, y_hbm, acc_out, cnt_out, src0, src1, dst0, dst1,
             rows0, rows1, sem0, sem1, ones_v, acc_sh, cnt_sh):
        cid = lax.axis_index("c")
        sid = lax.axis_index("s")
        wid = sid * NC + cid
        row0, is_last = _tile_layout(sid)

        def zrows(i, _):
            rows0[i, pl.ds(0, 16)] = jnp.zeros((16,), jnp.float32)
            rows0[i, pl.ds(16, 16)] = jnp.zeros((16,), jnp.float32)
            ones_v[i, pl.ds(0, 16)] = jnp.zeros((16,), jnp.float32)
            return 0

        lax.fori_loop(0, RLAST, zrows, 0)
        _copy_rows(lambda nr: rows0.at[pl.ds(0, nr)],
                   lambda nr: acc_sh.at[pl.ds(row0, nr)], is_last)
        _copy_rows(lambda nr: ones_v.at[pl.ds(0, nr)],
                   lambda nr: cnt_sh.at[pl.ds(row0, nr)], is_last)

        def fill_ones(i, _):
            ones_v[i, pl.ds(0, 16)] = jnp.ones((16,), jnp.float32)
            return 0

        lax.fori_loop(0, CH, fill_ones, 0)
        plsc.subcore_barrier()

        bufs = ((src0, dst0, rows0, sem0), (src1, dst1, rows1, sem1))

        def scatter(rv, dv):
            pltpu.sync_copy(rv, acc_sh.at[dv], add=True)
            pltpu.sync_copy(ones_v, cnt_sh.at[dv], add=True)

        _edge_pipeline(ei_hbm, wid, bufs, y_hbm, scatter)
        plsc.subcore_barrier()

        _copy_rows(lambda nr: acc_sh.at[pl.ds(row0, nr)],
                   lambda nr: acc_out.at[cid, pl.ds(row0, nr)], is_last)
        _copy_rows(lambda nr: cnt_sh.at[pl.ds(row0, nr)],
                   lambda nr: cnt_out.at[cid, pl.ds(row0, nr)], is_last)

    run = pl.kernel(
        body,
        out_type=[_f32(NC, N, HID), _f32(NC, N, CNTW)],
        mesh=_MESH,
        scratch_types=scratch, compiler_params=_SC_PARAMS,
        name="sage_sc_pass1",
    )
    return run(edge_index, y1)


def _sc_pass2(edge_index, acc1, cntp, xrb):
    scratch = [
        pltpu.VMEM((CH,), jnp.int32),
        pltpu.VMEM((CH,), jnp.int32),
        pltpu.VMEM((CH,), jnp.int32),
        pltpu.VMEM((CH,), jnp.int32),
        pltpu.VMEM((CH, HID), jnp.float32),
        pltpu.VMEM((CH, HID), jnp.float32),
        pltpu.SemaphoreType.DMA,
        pltpu.SemaphoreType.DMA,
        pltpu.VMEM((RLAST, CNTW), jnp.float32),
        pltpu.VMEM((RLAST, CNTW), jnp.float32),
        pltpu.VMEM((RLAST, HID), jnp.float32),
        pltpu.VMEM_SHARED((N, HID), jnp.float32),
    ]

    def body(ei_hbm, acc1_hbm, cntp_hbm, xrb_hbm, acc_out, h_out,
             src0, src1, dst0, dst1, rows0, rows1, sem0, sem1,
             cb0, cb1, xb, acc_sh):
        cid = lax.axis_index("c")
        sid = lax.axis_index("s")
        wid = sid * NC + cid
        row0, is_last = _tile_layout(sid)

        def stage(nr):
            ds = [
                pltpu.make_async_copy(acc1_hbm.at[0, pl.ds(row0, nr)],
                                      rows0.at[pl.ds(0, nr)], sem0),
                pltpu.make_async_copy(acc1_hbm.at[1, pl.ds(row0, nr)],
                                      rows1.at[pl.ds(0, nr)], sem0),
                pltpu.make_async_copy(cntp_hbm.at[0, pl.ds(row0, nr)],
                                      cb0.at[pl.ds(0, nr)], sem0),
                pltpu.make_async_copy(cntp_hbm.at[1, pl.ds(row0, nr)],
                                      cb1.at[pl.ds(0, nr)], sem0),
                pltpu.make_async_copy(xrb_hbm.at[pl.ds(row0, nr)],
                                      xb.at[pl.ds(0, nr)], sem0),
            ]
            for d in ds:
                d.start()
            for d in ds:
                d.wait()

        _per_tile(is_last, stage)

        def hrow(r, _):
            cv = jnp.maximum(cb0[r, pl.ds(0, 16)] + cb1[r, pl.ds(0, 16)], 1.0)
            inv = 1.0 / cv
            for j in range(HID // 16):
                sl = pl.ds(j * 16, 16)
                v = (rows0[r, sl] + rows1[r, sl]) * inv + xb[r, sl]
                xb[r, sl] = jnp.maximum(v, 0.0)
            return 0

        def hcompute(nr):
            lax.fori_loop(0, nr, hrow, 0)
        _per_tile(is_last, hcompute)

        _copy_rows(lambda nr: xb.at[pl.ds(0, nr)],
                   lambda nr: h_out.at[cid, pl.ds(row0, nr)], is_last)

        def zrows(i, _):
            rows0[i, pl.ds(0, 16)] = jnp.zeros((16,), jnp.float32)
            rows0[i, pl.ds(16, 16)] = jnp.zeros((16,), jnp.float32)
            return 0

        lax.fori_loop(0, RLAST, zrows, 0)
        _copy_rows(lambda nr: rows0.at[pl.ds(0, nr)],
                   lambda nr: acc_sh.at[pl.ds(row0, nr)], is_last)
        plsc.subcore_barrier()

        bufs = ((src0, dst0, rows0, sem0), (src1, dst1, rows1, sem1))

        def scatter(rv, dv):
            pltpu.sync_copy(rv, acc_sh.at[dv], add=True)

        _edge_pipeline(ei_hbm, wid, bufs, h_out.at[cid], scatter)
        plsc.subcore_barrier()

        def divrow(r, _):
            cv = jnp.maximum(cb0[r, pl.ds(0, 16)] + cb1[r, pl.ds(0, 16)], 1.0)
            inv = 1.0 / cv
            for j in range(HID // 16):
                sl = pl.ds(j * 16, 16)
                rows0[r, sl] = rows0[r, sl] * inv
            return 0

        def divphase(nr):
            pltpu.sync_copy(acc_sh.at[pl.ds(row0, nr)], rows0.at[pl.ds(0, nr)])
            lax.fori_loop(0, nr, divrow, 0)
            pltpu.sync_copy(rows0.at[pl.ds(0, nr)],
                            acc_out.at[cid, pl.ds(row0, nr)])

        _per_tile(is_last, divphase)

    run = pl.kernel(
        body,
        out_type=[_f32(NC, N, HID), _f32(NC, N, HID)],
        mesh=_MESH, scratch_types=scratch, compiler_params=_SC_PARAMS,
        name="sage_sc_pass2",
    )
    return run(edge_index, acc1, cntp, xrb)


def _dotT(a, w):
    return lax.dot_general(a, w, (((1,), (1,)), ((), ())),
                           preferred_element_type=jnp.float32)


def _pre_body(x_ref, wl_ref, wr_ref, b1_ref, y1_ref, xrb_ref):
    x = x_ref[...]
    y1_ref[...] = _dotT(x, wl_ref[...])
    xrb_ref[...] = _dotT(x, wr_ref[...]) + b1_ref[...]


def _out_body(acc2p_ref, h0p_ref, w2l_ref, w2r_ref, b2_ref, out_ref):
    mean2 = acc2p_ref[0] + acc2p_ref[1]
    h = h0p_ref[0]
    wl = w2l_ref[...]
    wr = w2r_ref[...]
    b2 = b2_ref[...]
    for i in range(4):
        sl = slice(32 * i, 32 * i + 32)
        out_ref[:, sl] = (_dotT(mean2[:, sl], wl) + b2
                          + _dotT(h[:, sl], wr))


def kernel(x, edge_index, W1_l, b1_l, W1_r, W2_l, b2_l, W2_r):
    y1, xrb = pl.pallas_call(
        _pre_body,
        out_shape=[_f32(N, HID), _f32(N, HID)],
    )(x, W1_l, W1_r, b1_l.reshape(1, HID))

    acc1, cntp = _sc_pass1(edge_index, y1)
    acc2, h2 = _sc_pass2(edge_index, acc1, cntp, xrb)

    acc2p = acc2.reshape(NC, N // 4, 128)
    h2p = h2.reshape(NC, N // 4, 128)

    outp = pl.pallas_call(
        _out_body,
        out_shape=_f32(N // 4, 128),
        grid=(1,),
        in_specs=[
            pl.BlockSpec((NC, N // 4, 128), lambda i: (0, 0, 0)),
            pl.BlockSpec((1, N // 4, 128), lambda i: (0, 0, 0)),
            pl.BlockSpec((HID, HID), lambda i: (0, 0)),
            pl.BlockSpec((HID, HID), lambda i: (0, 0)),
            pl.BlockSpec((1, HID), lambda i: (0, 0)),
        ],
        out_specs=pl.BlockSpec((N // 4, 128), lambda i: (0, 0)),
    )(acc2p, h2p, W2_l, W2_r, b2_l.reshape(1, HID))

    return outp.reshape(N, HID)
